# baseline (speedup 1.0000x reference)
import jax
import jax.numpy as jnp
from jax import lax
from jax.experimental import pallas as pl
from jax.experimental.pallas import tpu as pltpu
from jax.experimental.pallas import tpu_sc as plsc

N = 10000
E = 320000
IN_C = 256
D = 128
DH = 64
HOPS = 3

NC = 2
NS = 16
L = 16

NP = 10240
NROW = NP // NS
CH = 128
T32 = 10240
NCH32 = T32 // CH
EP = 32 * T32
T16 = EP // NS
NCH16 = T16 // CH
NBUF = 4

_i32 = jnp.int32
_f32 = jnp.float32


def _bcast16(v):
    return lax.broadcast_in_dim(jnp.asarray(v, _i32), (L,), ())


def _rsqrt16(d):
    i = plsc.bitcast(d, _i32)
    y = plsc.bitcast(jnp.asarray(0x5F3759DF, _i32) - lax.shift_right_arithmetic(i, 1), _f32)
    for _ in range(3):
        y = y * (1.5 - 0.5 * d * y * y)
    return jnp.where(d > 0.0, y, 0.0)


def _mm_body(x_ref, w_ref, b_ref, o_ref):
    acc = lax.dot_general(x_ref[...], w_ref[...], (((1,), (1,)), ((), ())),
                          preferred_element_type=_f32)
    o_ref[...] = acc + b_ref[...]


def _matmul(xp, W, b2d):
    BM = 1024
    return pl.pallas_call(
        _mm_body,
        grid=(NP // BM,),
        in_specs=[
            pl.BlockSpec((BM, IN_C), lambda m: (m, 0)),
            pl.BlockSpec((D, IN_C), lambda m: (0, 0)),
            pl.BlockSpec((1, D), lambda m: (0, 0)),
        ],
        out_specs=pl.BlockSpec((BM, D), lambda m: (m, 0)),
        out_shape=jax.ShapeDtypeStruct((NP, D), _f32),
    )(xp, W, b2d)


_MESH = plsc.VectorSubcoreMesh(core_axis_name="c", subcore_axis_name="s")
_SC_PARAMS = pltpu.CompilerParams(
    needs_layout_passes=False, use_tc_tiling_on_sc=False)


def _deg_body(col2_hbm, ew_hbm, degp0, degp1, cb2d, ewbuf, dbuf, degout,
              dacc, *vs):
    valbufs, ssems = vs[:NBUF], vs[NBUF:]
    c = lax.axis_index("c")
    s = lax.axis_index("s")
    t = c * NS + s

    pltpu.sync_copy(col2_hbm.at[pl.ds(t * NCH32, NCH32)], cb2d)
    pltpu.sync_copy(ew_hbm.at[pl.ds(t * T32, T32)], ewbuf)

    def _zrow(i, _):
        valbufs[0][i, :] = jnp.zeros((L,), _f32)
        return 0
    lax.fori_loop(0, CH, _zrow, 0)
    for kk in range(NROW // CH):
        pltpu.sync_copy(valbufs[0], dacc.at[pl.ds(s * NROW + kk * CH, CH)])
    plsc.subcore_barrier()

    def _fill(ch, b):
        def _f(j16, _):
            for l in range(L):
                j = j16 * L + l
                v = plsc.load_gather(ewbuf, [_bcast16(ch * CH + j)])
                valbufs[b][j, :] = v
            return 0
        lax.fori_loop(0, CH // L, _f, 0)

    def _scat(ch, b):
        return pltpu.make_async_copy(valbufs[b], dacc.at[cb2d.at[ch]], ssems[b])

    def _ring(i, _):
        for b in range(NBUF):
            ch = i * NBUF + b

            @pl.when(i > 0)
            def _():
                _scat(ch - NBUF, b).wait()

            _fill(ch, b)
            pltpu.async_copy(valbufs[b], dacc.at[cb2d.at[ch]], ssems[b],
                             add=True)
        return 0
    lax.fori_loop(0, NCH32 // NBUF, _ring, 0)
    for b in range(NBUF):
        _scat(NCH32 - NBUF + b, b).wait()
    plsc.subcore_barrier()

    pltpu.sync_copy(dacc.at[pl.ds(s * NROW, NROW)], dbuf)
    lanes0 = jnp.zeros((L,), _i32)

    def _extract(v, _):
        rows = v * L + lax.iota(_i32, L)
        degout[pl.ds(v * L, L)] = plsc.load_gather(dbuf, [rows, lanes0])
        return 0
    lax.fori_loop(0, NROW // L, _extract, 0)

    @pl.when(c == 0)
    def _():
        pltpu.sync_copy(degout, degp0.at[pl.ds(s * NROW, NROW)])

    @pl.when(c == 1)
    def _():
        pltpu.sync_copy(degout, degp1.at[pl.ds(s * NROW, NROW)])


_deg_kernel = pl.kernel(
    _deg_body,
    out_type=(jax.ShapeDtypeStruct((NP,), _f32),
              jax.ShapeDtypeStruct((NP,), _f32)),
    mesh=_MESH,
    scratch_types=(
        pltpu.VMEM((NCH32, CH), _i32),
        pltpu.VMEM((T32,), _f32),
        pltpu.VMEM((NROW, L), _f32),
        pltpu.VMEM((NROW,), _f32),
        pltpu.VMEM_SHARED((NP, L), _f32),
    ) + (pltpu.VMEM((CH, L), _f32),) * NBUF
      + (pltpu.SemaphoreType.DMA,) * NBUF,
    compiler_params=_SC_PARAMS,
)


def _w_body(row_hbm, col_hbm, ew_hbm, degp0, degp1, w_hbm, dp0, dp1, rbuf,
            cbuf, ebuf):
    c = lax.axis_index("c")
    s = lax.axis_index("s")
    tb = (c * NS + s) * T32

    pltpu.sync_copy(degp0, dp0)
    pltpu.sync_copy(degp1, dp1)

    def _dis(v, _):
        sl = pl.ds(v * L, L)
        dp0[sl] = _rsqrt16(dp0[sl] + dp1[sl])
        return 0
    lax.fori_loop(0, NP // L, _dis, 0)

    pltpu.sync_copy(row_hbm.at[pl.ds(tb, T32)], rbuf)
    pltpu.sync_copy(col_hbm.at[pl.ds(tb, T32)], cbuf)
    pltpu.sync_copy(ew_hbm.at[pl.ds(tb, T32)], ebuf)

    def _wgrp(g, _):
        sl = pl.ds(g * L, L)
        dr = plsc.load_gather(dp0, [rbuf[sl]])
        dc = plsc.load_gather(dp0, [cbuf[sl]])
        ebuf[sl] = dr * ebuf[sl] * dc
        return 0
    lax.fori_loop(0, T32 // L, _wgrp, 0)
    pltpu.sync_copy(ebuf, w_hbm.at[pl.ds(tb, T32)])


_w_kernel = pl.kernel(
    _w_body,
    out_type=jax.ShapeDtypeStruct((EP,), _f32),
    mesh=_MESH,
    scratch_types=(
        pltpu.VMEM((NP,), _f32),
        pltpu.VMEM((NP,), _f32),
        pltpu.VMEM((T32,), _i32),
        pltpu.VMEM((T32,), _i32),
        pltpu.VMEM((T32,), _f32),
    ),
    compiler_params=_SC_PARAMS,
)


def _hop_body(h0, h1, row2_hbm, col2_hbm, w2_hbm, out0, out1, rix2d, col2d,
              hacc, *vs):
    bufs = vs[:NBUF]
    wbufs = vs[NBUF:2 * NBUF]
    gsems = vs[2 * NBUF:3 * NBUF]
    wsems = vs[3 * NBUF:]
    c = lax.axis_index("c")
    s = lax.axis_index("s")

    pltpu.sync_copy(row2_hbm.at[pl.ds(s * NCH16, NCH16)], rix2d)
    pltpu.sync_copy(col2_hbm.at[pl.ds(s * NCH16, NCH16)], col2d)

    def _zrow(i, _):
        for q in range(DH // L):
            bufs[0][i, pl.ds(q * L, L)] = jnp.zeros((L,), _f32)
        return 0
    lax.fori_loop(0, CH, _zrow, 0)
    for kk in range(NROW // CH):
        pltpu.sync_copy(bufs[0], hacc.at[pl.ds(s * NROW + kk * CH, CH)])
    plsc.subcore_barrier()

    def _run(hsrc, odst):
        def _gath(ch, b):
            return pltpu.make_async_copy(hsrc.at[rix2d.at[ch]], bufs[b], gsems[b])

        def _scat_sync(ch, b):
            pltpu.sync_copy(bufs[b], hacc.at[col2d.at[ch]], add=True)

        def _wload(ch, b):
            return pltpu.make_async_copy(w2_hbm.at[s * NCH16 + ch], wbufs[b],
                                         wsems[b])

        def _scale(ch, b):
            def _sc(j16, _):
                for l in range(L):
                    j = j16 * L + l
                    ws = plsc.load_gather(wbufs[b], [_bcast16(j)])
                    for q in range(DH // L):
                        sl = pl.ds(q * L, L)
                        bufs[b][j, sl] = bufs[b][j, sl] * ws
                return 0
            lax.fori_loop(0, CH // L, _sc, 0)

        _gath(0, 0).start()
        _wload(0, 0).start()
        _gath(1, 1).start()
        _wload(1, 1).start()

        def _ring(i, _):
            for b in range(NBUF):
                ch = i * NBUF + b
                _gath(ch, b).wait()
                _wload(ch, b).wait()
                _scale(ch, b)
                _scat_sync(ch, b)
                b2 = (b + 2) % NBUF
                if b < 2:
                    _gath(ch + 2, b2).start()
                    _wload(ch + 2, b2).start()
                else:
                    @pl.when(i < NCH16 // NBUF - 1)
                    def _():
                        _gath(ch + 2, b2).start()
                        _wload(ch + 2, b2).start()
            return 0
        lax.fori_loop(0, NCH16 // NBUF, _ring, 0)
        plsc.subcore_barrier()
        sl = pl.ds(s * NROW, NROW)
        pltpu.sync_copy(hacc.at[sl], odst.at[sl])

    @pl.when(c == 0)
    def _():
        _run(h0, out0)

    @pl.when(c == 1)
    def _():
        _run(h1, out1)


_hop_kernel = pl.kernel(
    _hop_body,
    out_type=(jax.ShapeDtypeStruct((NP, DH), _f32),
              jax.ShapeDtypeStruct((NP, DH), _f32)),
    mesh=_MESH,
    scratch_types=(
        pltpu.VMEM((NCH16, CH), _i32),
        pltpu.VMEM((NCH16, CH), _i32),
        pltpu.VMEM_SHARED((NP, DH), _f32),
    ) + (pltpu.VMEM((CH, DH), _f32),) * NBUF
      + (pltpu.VMEM((CH,), _f32),) * NBUF
      + (pltpu.SemaphoreType.DMA,) * (2 * NBUF),
    compiler_params=_SC_PARAMS,
)


@jax.jit
def kernel(x, edge_index, edge_weight, W, b):
    row = jnp.pad(edge_index[0], (0, EP - E))
    col = jnp.pad(edge_index[1], (0, EP - E))
    ew = jnp.pad(edge_weight, (0, EP - E))
    xp = jnp.pad(x, ((0, NP - N), (0, 0)))

    h = _matmul(xp, W, b.reshape(1, D))
    row2 = row.reshape(EP // CH, CH)
    col2 = col.reshape(EP // CH, CH)
    degp0, degp1 = _deg_kernel(col2, ew)
    w = _w_kernel(row, col, ew, degp0, degp1)

    w2 = w.reshape(EP // CH, CH)
    h0 = h[:, :DH]
    h1 = h[:, DH:]
    for _ in range(HOPS):
        h0, h1 = _hop_kernel(h0, h1, row2, col2, w2)
    return jnp.concatenate([h0[:N], h1[:N]], axis=1)


# speedup vs baseline: 1.4191x; 1.4191x over previous
import jax
import jax.numpy as jnp
from jax import lax
from jax.experimental import pallas as pl
from jax.experimental.pallas import tpu as pltpu
from jax.experimental.pallas import tpu_sc as plsc

N = 10000
E = 320000
IN_C = 256
D = 128
DH = 64
HOPS = 3

NC = 2
NS = 16
L = 16

NP = 10240
NROW = NP // NS
CH = 128
T32 = 10240
NCH32 = T32 // CH
EP = 32 * T32
T16 = EP // NS
NCH16 = T16 // CH
NBUF = 4

_i32 = jnp.int32
_f32 = jnp.float32


def _bcast16(v):
    return lax.broadcast_in_dim(jnp.asarray(v, _i32), (L,), ())


def _rsqrt16(d):
    i = plsc.bitcast(d, _i32)
    y = plsc.bitcast(jnp.asarray(0x5F3759DF, _i32) - lax.shift_right_arithmetic(i, 1), _f32)
    for _ in range(3):
        y = y * (1.5 - 0.5 * d * y * y)
    return jnp.where(d > 0.0, y, 0.0)


def _mm_body(x_ref, w_ref, b_ref, o_ref):
    acc = lax.dot_general(x_ref[...], w_ref[...], (((1,), (1,)), ((), ())),
                          preferred_element_type=_f32)
    o_ref[...] = acc + b_ref[...]


def _matmul(xp, W, b2d):
    BM = 1024
    return pl.pallas_call(
        _mm_body,
        grid=(NP // BM,),
        in_specs=[
            pl.BlockSpec((BM, IN_C), lambda m: (m, 0)),
            pl.BlockSpec((D, IN_C), lambda m: (0, 0)),
            pl.BlockSpec((1, D), lambda m: (0, 0)),
        ],
        out_specs=pl.BlockSpec((BM, D), lambda m: (m, 0)),
        out_shape=jax.ShapeDtypeStruct((NP, D), _f32),
    )(xp, W, b2d)


_MESH = plsc.VectorSubcoreMesh(core_axis_name="c", subcore_axis_name="s")
_SC_PARAMS = pltpu.CompilerParams(
    needs_layout_passes=False, use_tc_tiling_on_sc=False)


def _deg_body(col2_hbm, ew_hbm, degp0, degp1, cb2d, ewbuf, dbuf, degout,
              dacc, *vs):
    valbufs, ssems = vs[:NBUF], vs[NBUF:]
    c = lax.axis_index("c")
    s = lax.axis_index("s")
    t = c * NS + s

    pltpu.sync_copy(col2_hbm.at[pl.ds(t * NCH32, NCH32)], cb2d)
    pltpu.sync_copy(ew_hbm.at[pl.ds(t * T32, T32)], ewbuf)

    def _zrow(i, _):
        valbufs[0][i, :] = jnp.zeros((L,), _f32)
        return 0
    lax.fori_loop(0, CH, _zrow, 0)
    for kk in range(NROW // CH):
        pltpu.sync_copy(valbufs[0], dacc.at[pl.ds(s * NROW + kk * CH, CH)])
    plsc.subcore_barrier()

    def _fill(ch, b):
        def _f(j16, _):
            for l in range(L):
                j = j16 * L + l
                v = plsc.load_gather(ewbuf, [_bcast16(ch * CH + j)])
                valbufs[b][j, :] = v
            return 0
        lax.fori_loop(0, CH // L, _f, 0)

    def _chunk(ch, _):
        _fill(ch, 0)
        pltpu.sync_copy(valbufs[0], dacc.at[cb2d.at[ch]], add=True)
        return 0
    lax.fori_loop(0, NCH32, _chunk, 0)
    plsc.subcore_barrier()

    pltpu.sync_copy(dacc.at[pl.ds(s * NROW, NROW)], dbuf)
    lanes0 = jnp.zeros((L,), _i32)

    def _extract(v, _):
        rows = v * L + lax.iota(_i32, L)
        degout[pl.ds(v * L, L)] = plsc.load_gather(dbuf, [rows, lanes0])
        return 0
    lax.fori_loop(0, NROW // L, _extract, 0)

    @pl.when(c == 0)
    def _():
        pltpu.sync_copy(degout, degp0.at[pl.ds(s * NROW, NROW)])

    @pl.when(c == 1)
    def _():
        pltpu.sync_copy(degout, degp1.at[pl.ds(s * NROW, NROW)])


_deg_kernel = pl.kernel(
    _deg_body,
    out_type=(jax.ShapeDtypeStruct((NP,), _f32),
              jax.ShapeDtypeStruct((NP,), _f32)),
    mesh=_MESH,
    scratch_types=(
        pltpu.VMEM((NCH32, CH), _i32),
        pltpu.VMEM((T32,), _f32),
        pltpu.VMEM((NROW, L), _f32),
        pltpu.VMEM((NROW,), _f32),
        pltpu.VMEM_SHARED((NP, L), _f32),
    ) + (pltpu.VMEM((CH, L), _f32),) * NBUF
      + (pltpu.SemaphoreType.DMA,) * NBUF,
    compiler_params=_SC_PARAMS,
)


def _w_body(row_hbm, col_hbm, ew_hbm, degp0, degp1, w_hbm, dp0, dp1, rbuf,
            cbuf, ebuf):
    c = lax.axis_index("c")
    s = lax.axis_index("s")
    tb = (c * NS + s) * T32

    pltpu.sync_copy(degp0, dp0)
    pltpu.sync_copy(degp1, dp1)

    def _dis(v, _):
        sl = pl.ds(v * L, L)
        dp0[sl] = _rsqrt16(dp0[sl] + dp1[sl])
        return 0
    lax.fori_loop(0, NP // L, _dis, 0)

    pltpu.sync_copy(row_hbm.at[pl.ds(tb, T32)], rbuf)
    pltpu.sync_copy(col_hbm.at[pl.ds(tb, T32)], cbuf)
    pltpu.sync_copy(ew_hbm.at[pl.ds(tb, T32)], ebuf)

    def _wgrp(g, _):
        sl = pl.ds(g * L, L)
        dr = plsc.load_gather(dp0, [rbuf[sl]])
        dc = plsc.load_gather(dp0, [cbuf[sl]])
        ebuf[sl] = dr * ebuf[sl] * dc
        return 0
    lax.fori_loop(0, T32 // L, _wgrp, 0)
    pltpu.sync_copy(ebuf, w_hbm.at[pl.ds(tb, T32)])


_w_kernel = pl.kernel(
    _w_body,
    out_type=jax.ShapeDtypeStruct((EP,), _f32),
    mesh=_MESH,
    scratch_types=(
        pltpu.VMEM((NP,), _f32),
        pltpu.VMEM((NP,), _f32),
        pltpu.VMEM((T32,), _i32),
        pltpu.VMEM((T32,), _i32),
        pltpu.VMEM((T32,), _f32),
    ),
    compiler_params=_SC_PARAMS,
)


def _hop_body(h0, h1, pk2_hbm, w_hbm, out0, out1, pk2d, wbuf, hacc, *vs):
    bufs = vs[:NBUF]
    ridx = vs[NBUF:2 * NBUF]
    cidx = vs[2 * NBUF:3 * NBUF]
    gsems = vs[3 * NBUF:4 * NBUF]
    ssems = vs[4 * NBUF:]
    c = lax.axis_index("c")
    s = lax.axis_index("s")

    pltpu.sync_copy(pk2_hbm.at[pl.ds(s * NCH16, NCH16)], pk2d)
    pltpu.sync_copy(w_hbm.at[pl.ds(s * T16, T16)], wbuf)

    def _zrow(i, _):
        for q in range(DH // L):
            bufs[0][i, pl.ds(q * L, L)] = jnp.zeros((L,), _f32)
        return 0
    lax.fori_loop(0, CH, _zrow, 0)
    for kk in range(NROW // CH):
        pltpu.sync_copy(bufs[0], hacc.at[pl.ds(s * NROW + kk * CH, CH)])
    plsc.subcore_barrier()

    def _unpack(ch, b):
        def _u(j16, _):
            sl = pl.ds(j16 * L, L)
            v = pk2d[ch, sl]
            ridx[b][sl] = lax.shift_right_logical(v, 16)
            cidx[b][sl] = lax.bitwise_and(v, jnp.asarray(0xFFFF, _i32))
            return 0
        lax.fori_loop(0, CH // L, _u, 0)

    def _run(hsrc, odst):
        def _gath(ch, b):
            return pltpu.make_async_copy(hsrc.at[ridx[b]], bufs[b], gsems[b])

        def _swait(b):
            return pltpu.make_async_copy(bufs[b], hacc.at[cidx[b]], ssems[b])

        def _scale(ch, b):
            def _sc(j16, _):
                for l in range(L):
                    j = j16 * L + l
                    ws = plsc.load_gather(wbuf, [_bcast16(ch * CH + j)])
                    for q in range(DH // L):
                        sl = pl.ds(q * L, L)
                        bufs[b][j, sl] = bufs[b][j, sl] * ws
                return 0
            lax.fori_loop(0, CH // L, _sc, 0)

        _unpack(0, 0)
        _gath(0, 0).start()
        _unpack(1, 1)
        _gath(1, 1).start()

        def _ring(i, _):
            for b in range(NBUF):
                ch = i * NBUF + b
                _gath(ch, b).wait()
                _scale(ch, b)
                pltpu.async_copy(bufs[b], hacc.at[cidx[b]], ssems[b], add=True)
                b2 = (b + 2) % NBUF
                if b < 2:
                    @pl.when(i > 0)
                    def _():
                        _swait(b2).wait()

                    _unpack(ch + 2, b2)
                    _gath(ch + 2, b2).start()
                else:
                    @pl.when(ch + 2 < NCH16)
                    def _():
                        _swait(b2).wait()
                        _unpack(ch + 2, b2)
                        _gath(ch + 2, b2).start()
            return 0
        lax.fori_loop(0, NCH16 // NBUF, _ring, 0)
        for ch in range(NCH16 - NBUF, NCH16):
            _swait(ch % NBUF).wait()
        plsc.subcore_barrier()
        sl = pl.ds(s * NROW, NROW)
        pltpu.sync_copy(hacc.at[sl], odst.at[sl])

    @pl.when(c == 0)
    def _():
        _run(h0, out0)

    @pl.when(c == 1)
    def _():
        _run(h1, out1)


_hop_kernel = pl.kernel(
    _hop_body,
    out_type=(jax.ShapeDtypeStruct((NP, DH), _f32),
              jax.ShapeDtypeStruct((NP, DH), _f32)),
    mesh=_MESH,
    scratch_types=(
        pltpu.VMEM((NCH16, CH), _i32),
        pltpu.VMEM((T16,), _f32),
        pltpu.VMEM_SHARED((NP, DH), _f32),
    ) + (pltpu.VMEM((CH, DH), _f32),) * NBUF
      + (pltpu.VMEM((CH,), _i32),) * (2 * NBUF)
      + (pltpu.SemaphoreType.DMA,) * (2 * NBUF),
    compiler_params=_SC_PARAMS,
)


@jax.jit
def kernel(x, edge_index, edge_weight, W, b):
    row = jnp.pad(edge_index[0], (0, EP - E))
    col = jnp.pad(edge_index[1], (0, EP - E))
    ew = jnp.pad(edge_weight, (0, EP - E))
    xp = jnp.pad(x, ((0, NP - N), (0, 0)))

    h = _matmul(xp, W, b.reshape(1, D))
    col2 = col.reshape(EP // CH, CH)
    degp0, degp1 = _deg_kernel(col2, ew)
    w = _w_kernel(row, col, ew, degp0, degp1)

    pk2 = ((row << 16) | col).reshape(EP // CH, CH)
    h0 = h[:, :DH]
    h1 = h[:, DH:]
    for _ in range(HOPS):
        h0, h1 = _hop_kernel(h0, h1, pk2, w)
    return jnp.concatenate([h0[:N], h1[:N]], axis=1)
